# Initial kernel scaffold; baseline (speedup 1.0000x reference)
import jax
import jax.numpy as jnp
from jax.experimental import pallas as pl


def kernel(x, edge_index, edge_attr, batch, atom_emb, bond_emb, t, W1, b1, bn_g, bn_b, W2, b2, ln_g, ln_b, Wp, bp):
    raise NotImplementedError("write your pallas kernel here")


# speedup vs baseline: 1.8739x; 1.8739x over previous
import jax
import jax.numpy as jnp
from jax.experimental import pallas as pl

N_NODES = 50000
N_GRAPHS = 128
EPS_GEN = 1e-7
EPS_BN = 1e-5
N_LAYERS = 7
N_ATOM_FEATS = 4
N_BOND_FEATS = 3
EMB = 64


def _bn(x, g, b):
    mu = jnp.mean(x, axis=0)
    var = jnp.var(x, axis=0)
    return (x - mu) * jax.lax.rsqrt(var + EPS_BN) * g + b


def _genconv(h, src, dst, eattr, t, W1, b1, bn_g, bn_b, W2, b2):
    msg = jax.nn.relu(h[src] + eattr) + EPS_GEN
    e = jnp.exp(msg * t)
    d = jax.ops.segment_sum(e, dst, num_segments=N_NODES)
    s = jax.ops.segment_sum(msg * e, dst, num_segments=N_NODES)
    agg = s / (d + 1e-16)
    out = agg + h
    z = out @ W1 + b1
    z = jax.nn.relu(_bn(z, bn_g, bn_b))
    return z @ W2 + b2


def _ident_body(x_ref, o_ref):
    o_ref[...] = x_ref[...]


def kernel(x, edge_index, edge_attr, batch, atom_emb, bond_emb, t, W1, b1,
           bn_g, bn_b, W2, b2, ln_g, ln_b, Wp, bp):
    h = jnp.zeros((x.shape[0], EMB), jnp.float32)
    for i in range(N_ATOM_FEATS):
        h = h + atom_emb[i][x[:, i]]
    ea = jnp.zeros((edge_attr.shape[0], EMB), jnp.float32)
    for i in range(N_BOND_FEATS):
        ea = ea + bond_emb[i][edge_attr[:, i]]
    src = edge_index[0]
    dst = edge_index[1]
    h = _genconv(h, src, dst, ea, t[0], W1[0], b1[0], bn_g[0], bn_b[0], W2[0], b2[0])
    for i in range(1, N_LAYERS):
        g = jax.nn.relu(_bn(h, ln_g[i], ln_b[i]))
        h = h + _genconv(g, src, dst, ea, t[i], W1[i], b1[i], bn_g[i], bn_b[i], W2[i], b2[i])
    h = _bn(h, ln_g[0], ln_b[0])
    cnt = jax.ops.segment_sum(jnp.ones((h.shape[0],), jnp.float32), batch, num_segments=N_GRAPHS)
    summed = jax.ops.segment_sum(h, batch, num_segments=N_GRAPHS)
    gm = summed / jnp.maximum(cnt, 1.0)[:, None]
    out = gm @ Wp + bp
    return pl.pallas_call(
        _ident_body,
        out_shape=jax.ShapeDtypeStruct(out.shape, out.dtype),
    )(out)


# speedup vs baseline: 6.9000x; 3.6822x over previous
import functools

import jax
import jax.numpy as jnp
from jax import lax
from jax.experimental import pallas as pl
from jax.experimental.pallas import tpu as pltpu
from jax.experimental.pallas import tpu_sc as plsc

N_NODES = 50000
N_EDGES = 800000
EMB = 64
HID = 128
N_LAYERS = 7
N_GRAPHS = 128
EPS_GEN = 1e-7
EPS_BN = 1e-5

NC = 2
NS = 16
NQ = 3125
NPART = 16
NQP = 3200
RPW = NQP // NS
CH = 256
EPAD = 800512
BLK = 512
NBLK = 98
EBLK = 1563
EA_ROWS = EBLK * BLK


def _lane_extract(vec, lane):
    lanes = lax.broadcasted_iota(jnp.int32, (16,), 0)
    s = jnp.sum(jnp.where(lanes == lane, vec, 0.0))
    return s.astype(jnp.int32)


def _sc_edge_body(h_hbm, ea_hbm, srcs_hbm, dsts_hbm, meta_hbm, par_hbm,
                  acc_hbm, meta_v, par_v, src_i, dst_i, idx_i, rows, eab, vb,
                  acc, sem0, sem1, sem2):
    cid = lax.axis_index("c")
    sid = lax.axis_index("s")
    pltpu.sync_copy(meta_hbm, meta_v)
    pltpu.sync_copy(par_hbm, par_v)
    Ac = [par_v[c, :] for c in range(4)]
    Bc = [par_v[4 + c, :] for c in range(4)]
    tv = par_v[8, :]
    lov = par_v[9, :]
    iota16 = lax.broadcasted_iota(jnp.int32, (16,), 0)
    sems = (sem0, sem1, sem2)

    for kq in range(NPART // NC):
        q = cid * (NPART // NC) + kq
        def _zvb(i, carry):
            for k in range(2):
                for c in range(8):
                    vb[k, i, pl.ds(c * 16, 16)] = jnp.zeros((16,), jnp.float32)
            return carry
        lax.fori_loop(0, 128, _zvb, 0)
        for off in range(0, RPW, 128):
            sz = min(128, RPW - off)
            pltpu.sync_copy(vb.at[0, pl.ds(0, sz)],
                            acc.at[pl.ds(sid * RPW + off, sz)])
        plsc.subcore_barrier()

        ws = meta_v[pl.ds(q * 32 + sid, 16)][0].astype(jnp.int32)
        we = meta_v[pl.ds(q * 32 + 16 + sid, 16)][0].astype(jnp.int32)
        qb = q * NQ
        nch = (we - ws + CH - 1) // CH

        def _chunk(j, carry):
            e0 = pl.multiple_of(ws + j * CH, 8)
            for k in range(2):
                pltpu.sync_copy(srcs_hbm.at[pl.ds(e0 + k * 128, 128)],
                                src_i.at[k])
                pltpu.sync_copy(dsts_hbm.at[pl.ds(e0 + k * 128, 128)],
                                dst_i.at[k])
            cps = []
            for k in range(2):
                cps.append(pltpu.async_copy(h_hbm.at[src_i.at[k]],
                                            rows.at[k], sems[k]))
            cps.append(pltpu.async_copy(ea_hbm.at[pl.ds(e0, CH)],
                                        eab, sems[2]))
            for cp in cps:
                cp.wait()
            for k in range(2):
                for v in range(8):
                    d16 = dst_i[k, pl.ds(v * 16, 16)]
                    dl = d16 - qb
                    eidx = (e0 + k * 128 + v * 16) + iota16
                    ok = (dl >= 0) & (dl < NQ) & (eidx < we)
                    idx_i[k, pl.ds(v * 16, 16)] = jnp.where(ok, dl, NQ)

            def _vals(i, carry2):
                for k in range(2):
                    for c in range(4):
                        r = rows[k, i, pl.ds(c * 16, 16)]
                        a = eab[k * 128 + i, pl.ds(c * 16, 16)]
                        gv = r * Ac[c] + Bc[c]
                        g = jnp.maximum(gv, lov * gv)
                        m = jnp.maximum(g + a, 0.0) + EPS_GEN
                        ev = jnp.exp(tv * m)
                        vb[k, i, pl.ds(c * 16, 16)] = ev
                        vb[k, i, pl.ds(64 + c * 16, 16)] = m * ev
                return carry2
            lax.fori_loop(0, 128, _vals, 0)
            for k in range(2):
                pltpu.sync_copy(vb.at[k], acc.at[idx_i.at[k]], add=True)
            return carry
        lax.fori_loop(0, nch, _chunk, 0)
        plsc.subcore_barrier()

        for off in range(0, RPW, 128):
            sz = min(128, RPW - off)
            pltpu.sync_copy(
                acc.at[pl.ds(sid * RPW + off, sz)],
                acc_hbm.at[q, pl.ds(sid * RPW + off, sz)])
        plsc.subcore_barrier()


_sc_edge = pl.kernel(
    _sc_edge_body,
    out_type=jax.ShapeDtypeStruct((NPART, NQP, 128), jnp.float32),
    mesh=plsc.VectorSubcoreMesh(core_axis_name="c", subcore_axis_name="s"),
    scratch_types=[
        pltpu.VMEM((544,), jnp.float32),
        pltpu.VMEM((16, 16), jnp.float32),
        pltpu.VMEM((2, 128), jnp.int32),
        pltpu.VMEM((2, 128), jnp.int32),
        pltpu.VMEM((2, 128), jnp.int32),
        pltpu.VMEM((2, 128, 128), jnp.float32),
        pltpu.VMEM((256, 64), jnp.float32),
        pltpu.VMEM((2, 128, 128), jnp.float32),
        pltpu.VMEM_SHARED((NQP, 128), jnp.float32),
        pltpu.SemaphoreType.DMA,
        pltpu.SemaphoreType.DMA,
        pltpu.SemaphoreType.DMA,
    ],
)


def _rowmask(i, n_rows, total):
    rows = i * BLK + lax.broadcasted_iota(jnp.int32, (n_rows, 1), 0)
    return rows < total


def _enc_atom_body(x0, x1, x2, x3, t0, t1, t2, t3, h0_ref):
    xs = (x0, x1, x2, x3)
    ts = (t0, t1, t2, t3)
    acc = jnp.zeros((BLK, EMB), jnp.float32)
    lanes = lax.broadcasted_iota(jnp.int32, (BLK, 128), 1)
    for k in range(4):
        xv = xs[k][0, 0, :]
        oh = (lanes == xv[:, None]).astype(jnp.float32)
        acc = acc + jnp.dot(oh, ts[k][...],
                            preferred_element_type=jnp.float32, precision=lax.Precision.HIGHEST)
    h0_ref[...] = jnp.concatenate(
        [acc, jnp.zeros((BLK, 128 - EMB), jnp.float32)], axis=1)


_enc_atom = pl.pallas_call(
    _enc_atom_body,
    grid=(NBLK,),
    in_specs=[pl.BlockSpec((1, 1, BLK), lambda i: (i, 0, 0))] * 4
    + [pl.BlockSpec((128, EMB), lambda i: (0, 0))] * 4,
    out_specs=pl.BlockSpec((BLK, 128), lambda i: (i, 0)),
    out_shape=jax.ShapeDtypeStruct((N_NODES, 128), jnp.float32),
)


def _enc_bond_body(a0, a1, a2, t0, t1, t2, ea_ref):
    As = (a0, a1, a2)
    ts = (t0, t1, t2)
    acc = jnp.zeros((BLK, EMB), jnp.float32)
    lanes = lax.broadcasted_iota(jnp.int32, (BLK, 16), 1)
    for k in range(3):
        xv = As[k][0, 0, :]
        oh = (lanes == xv[:, None]).astype(jnp.float32)
        acc = acc + jnp.dot(oh, ts[k][...],
                            preferred_element_type=jnp.float32, precision=lax.Precision.HIGHEST)
    ea_ref[...] = acc


_enc_bond = pl.pallas_call(
    _enc_bond_body,
    grid=(EBLK,),
    in_specs=[pl.BlockSpec((1, 1, BLK), lambda i: (i, 0, 0))] * 3
    + [pl.BlockSpec((16, EMB), lambda i: (0, 0))] * 3,
    out_specs=pl.BlockSpec((BLK, EMB), lambda i: (i, 0)),
    out_shape=jax.ShapeDtypeStruct((EA_ROWS, EMB), jnp.float32),
)


def _mm1_body(acc_ref, h_ref, W1_ref, b1_ref, A_ref, B_ref, lo_ref,
              z_ref, ps_ref, pss_ref):
    i = pl.program_id(0)
    accv = acc_ref[...]
    h = h_ref[:, :EMB]
    v = h * A_ref[...] + B_ref[...]
    g = jnp.maximum(v, lo_ref[...] * v)
    d = accv[:, :EMB]
    s = accv[:, EMB:]
    out = s / (d + 1e-16) + g
    z = jnp.dot(out, W1_ref[...], preferred_element_type=jnp.float32)
    z = z + b1_ref[...]
    z = jnp.where(_rowmask(i, BLK, N_NODES), z, 0.0)
    z_ref[...] = z
    ps_ref[...] = jnp.sum(z, axis=0, keepdims=True)[None]
    pss_ref[...] = jnp.sum(z * z, axis=0, keepdims=True)[None]


_mm1 = pl.pallas_call(
    _mm1_body,
    grid=(NBLK,),
    in_specs=[
        pl.BlockSpec((BLK, 128), lambda i: (i, 0)),
        pl.BlockSpec((BLK, 128), lambda i: (i, 0)),
        pl.BlockSpec((EMB, HID), lambda i: (0, 0)),
        pl.BlockSpec((1, HID), lambda i: (0, 0)),
        pl.BlockSpec((1, EMB), lambda i: (0, 0)),
        pl.BlockSpec((1, EMB), lambda i: (0, 0)),
        pl.BlockSpec((1, EMB), lambda i: (0, 0)),
    ],
    out_specs=[
        pl.BlockSpec((BLK, HID), lambda i: (i, 0)),
        pl.BlockSpec((1, 1, HID), lambda i: (i, 0, 0)),
        pl.BlockSpec((1, 1, HID), lambda i: (i, 0, 0)),
    ],
    out_shape=[
        jax.ShapeDtypeStruct((N_NODES, HID), jnp.float32),
        jax.ShapeDtypeStruct((NBLK, 1, HID), jnp.float32),
        jax.ShapeDtypeStruct((NBLK, 1, HID), jnp.float32),
    ],
)


def _mm2_body(z_ref, h_ref, zA_ref, zB_ref, W2_ref, b2_ref, rm_ref,
              h2_ref, hs_ref, hss_ref):
    i = pl.program_id(0)
    z = z_ref[...]
    zn = jnp.maximum(z * zA_ref[...] + zB_ref[...], 0.0)
    y = jnp.dot(zn, W2_ref[...], preferred_element_type=jnp.float32)
    y = y + b2_ref[...]
    h2 = rm_ref[...] * h_ref[:, :EMB] + y
    h2 = jnp.where(_rowmask(i, BLK, N_NODES), h2, 0.0)
    h2_ref[...] = jnp.concatenate(
        [h2, jnp.zeros((BLK, 128 - EMB), jnp.float32)], axis=1)
    hs_ref[...] = jnp.sum(h2, axis=0, keepdims=True)[None]
    hss_ref[...] = jnp.sum(h2 * h2, axis=0, keepdims=True)[None]


_mm2 = pl.pallas_call(
    _mm2_body,
    grid=(NBLK,),
    in_specs=[
        pl.BlockSpec((BLK, HID), lambda i: (i, 0)),
        pl.BlockSpec((BLK, 128), lambda i: (i, 0)),
        pl.BlockSpec((1, HID), lambda i: (0, 0)),
        pl.BlockSpec((1, HID), lambda i: (0, 0)),
        pl.BlockSpec((HID, EMB), lambda i: (0, 0)),
        pl.BlockSpec((1, EMB), lambda i: (0, 0)),
        pl.BlockSpec((1, EMB), lambda i: (0, 0)),
    ],
    out_specs=[
        pl.BlockSpec((BLK, 128), lambda i: (i, 0)),
        pl.BlockSpec((1, 1, EMB), lambda i: (i, 0, 0)),
        pl.BlockSpec((1, 1, EMB), lambda i: (i, 0, 0)),
    ],
    out_shape=[
        jax.ShapeDtypeStruct((N_NODES, 128), jnp.float32),
        jax.ShapeDtypeStruct((NBLK, 1, EMB), jnp.float32),
        jax.ShapeDtypeStruct((NBLK, 1, EMB), jnp.float32),
    ],
)


def _pool_body(h_ref, b_ref, A_ref, B_ref, Wp_ref, bp_ref, o_ref,
               pooled, cnt):
    i = pl.program_id(0)

    @pl.when(i == 0)
    def _():
        pooled[...] = jnp.zeros_like(pooled)
        cnt[...] = jnp.zeros_like(cnt)

    hn = h_ref[:, :EMB] * A_ref[...] + B_ref[...]
    hn = jnp.where(_rowmask(i, BLK, N_NODES), hn, 0.0)
    bv = b_ref[0, 0, :]
    gi = lax.broadcasted_iota(jnp.int32, (N_GRAPHS, BLK), 0)
    P = (gi == bv[None, :]).astype(jnp.float32)
    pooled[...] += jnp.dot(P, hn, preferred_element_type=jnp.float32, precision=lax.Precision.HIGHEST)
    cnt[...] += jnp.dot(P, jnp.ones((BLK, EMB), jnp.float32),
                        preferred_element_type=jnp.float32, precision=lax.Precision.HIGHEST)

    @pl.when(i == NBLK - 1)
    def _():
        gm = pooled[...] / jnp.maximum(cnt[...], 1.0)
        o_ref[...] = jnp.dot(gm, Wp_ref[...],
                             preferred_element_type=jnp.float32, precision=lax.Precision.HIGHEST) + bp_ref[...]


_pool = pl.pallas_call(
    _pool_body,
    grid=(NBLK,),
    in_specs=[
        pl.BlockSpec((BLK, 128), lambda i: (i, 0)),
        pl.BlockSpec((1, 1, BLK), lambda i: (i, 0, 0)),
        pl.BlockSpec((1, EMB), lambda i: (0, 0)),
        pl.BlockSpec((1, EMB), lambda i: (0, 0)),
        pl.BlockSpec((EMB, 128), lambda i: (0, 0)),
        pl.BlockSpec((1, 128), lambda i: (0, 0)),
    ],
    out_specs=pl.BlockSpec((N_GRAPHS, 128), lambda i: (0, 0)),
    out_shape=jax.ShapeDtypeStruct((N_GRAPHS, 128), jnp.float32),
    scratch_shapes=[
        pltpu.VMEM((N_GRAPHS, EMB), jnp.float32),
        pltpu.VMEM((N_GRAPHS, EMB), jnp.float32),
    ],
)


def _pad_to(a, n, fill):
    return jnp.concatenate(
        [a, jnp.full((n - a.shape[0],) + a.shape[1:], fill, a.dtype)])


def _blocks3d(col, nblk, fill):
    return _pad_to(col, nblk * BLK, fill).reshape(nblk, BLK)[:, None, :]


def kernel(x, edge_index, edge_attr, batch, atom_emb, bond_emb, t, W1, b1,
           bn_g, bn_b, W2, b2, ln_g, ln_b, Wp, bp):
    f32 = jnp.float32
    src = edge_index[0].astype(jnp.int32)
    dst = edge_index[1].astype(jnp.int32)
    perm = jnp.argsort(dst)
    dsts = dst[perm]
    srcs = src[perm]
    attr_s = edge_attr[perm].astype(jnp.int32)

    srcs_p = _pad_to(srcs, EPAD, 0)
    dsts_p = _pad_to(dsts, EPAD, 1 << 29)

    qb = jnp.searchsorted(
        dsts, jnp.arange(0, 50001, NQ, dtype=jnp.int32)).astype(jnp.int32)
    w = jnp.arange(NS, dtype=jnp.int32)
    lo_q = qb[:NPART][:, None]
    hi_q = qb[1:][:, None]
    starts = (lo_q + (w[None, :] * (hi_q - lo_q)) // NS) & ~7
    ends = jnp.concatenate([starts[:, 1:], hi_q], axis=1)
    meta = jnp.concatenate(
        [jnp.stack([starts, ends], axis=1).astype(jnp.float32).reshape(
            NPART * 32),
         jnp.zeros((32,), jnp.float32)])
    meta = _pad_to(meta, 544, 0.0)

    xT = [_blocks3d(x[:, k].astype(jnp.int32), NBLK, 0) for k in range(4)]
    h = _enc_atom(*xT, atom_emb[0], atom_emb[1], atom_emb[2], atom_emb[3])
    aT = [_blocks3d(attr_s[:, k], EBLK, 0) for k in range(3)]
    ea_s = _enc_bond(*aT, bond_emb[0], bond_emb[1], bond_emb[2])

    ones64 = jnp.ones((EMB,), f32)
    zeros64 = jnp.zeros((EMB,), f32)
    A, B, lo = ones64, zeros64, 1.0

    for i in range(N_LAYERS):
        par = jnp.zeros((16, 16), f32)
        par = par.at[0:4, :].set(A.reshape(4, 16))
        par = par.at[4:8, :].set(B.reshape(4, 16))
        par = par.at[8, :].set(t[i])
        par = par.at[9, :].set(lo)
        acc4 = _sc_edge(h, ea_s, srcs_p, dsts_p, meta, par)
        acc = acc4[:, :NQ, :].reshape(N_NODES, 128)
        z, ps, pss = _mm1(acc, h, W1[i], b1[i].reshape(1, -1),
                          A.reshape(1, -1), B.reshape(1, -1),
                          jnp.full((1, EMB), lo, f32))
        mu = jnp.sum(ps, (0, 1)) / N_NODES
        var = jnp.sum(pss, (0, 1)) / N_NODES - mu * mu
        zA = bn_g[i] * lax.rsqrt(var + EPS_BN)
        zB = bn_b[i] - mu * zA
        rm = jnp.full((1, EMB), 0.0 if i == 0 else 1.0, f32)
        h, hs, hss = _mm2(z, h, zA.reshape(1, -1), zB.reshape(1, -1),
                          W2[i], b2[i].reshape(1, -1), rm)
        muh = jnp.sum(hs, (0, 1)) / N_NODES
        varh = jnp.sum(hss, (0, 1)) / N_NODES - muh * muh
        j = i + 1 if i + 1 < N_LAYERS else 0
        A = ln_g[j] * lax.rsqrt(varh + EPS_BN)
        B = ln_b[j] - muh * A
        lo = 0.0

    b3d = _blocks3d(batch.astype(jnp.int32), NBLK, 1 << 29)
    Wp_pad = jnp.zeros((EMB, 128), f32).at[:, :Wp.shape[1]].set(Wp)
    bp_pad = jnp.zeros((1, 128), f32).at[0, :bp.shape[0]].set(bp)
    outp = _pool(h, b3d, A.reshape(1, -1), B.reshape(1, -1), Wp_pad, bp_pad)
    return outp[:, :Wp.shape[1]]


# speedup vs baseline: 9.3606x; 1.3566x over previous
import functools

import jax
import jax.numpy as jnp
from jax import lax
from jax.experimental import pallas as pl
from jax.experimental.pallas import tpu as pltpu
from jax.experimental.pallas import tpu_sc as plsc

N_NODES = 50000
N_EDGES = 800000
EMB = 64
HID = 128
N_LAYERS = 7
N_GRAPHS = 128
EPS_GEN = 1e-7
EPS_BN = 1e-5

NC = 2
NS = 16
NQ = 2500
NPART = 20
NQP = 2560
RPW = NQP // NS
CH = 128
EPAD = 800512
BLK = 512
NBLK = 98
EBLK = 1564
EA_ROWS = EBLK * BLK


def _lane_extract(vec, lane):
    lanes = lax.broadcasted_iota(jnp.int32, (16,), 0)
    s = jnp.sum(jnp.where(lanes == lane, vec, 0.0))
    return s.astype(jnp.int32)


def _sc_edge_body(h_hbm, ea_hbm, sd_hbm, meta_hbm, par_hbm,
                  acc_hbm, meta_v, par_v, sd_i, idx_i, rows, eab, vb, zbuf,
                  acc, semg0, semg1, seme0, seme1):
    cid = lax.axis_index("c")
    sid = lax.axis_index("s")
    pltpu.sync_copy(meta_hbm, meta_v)
    pltpu.sync_copy(par_hbm, par_v)
    Ac = [par_v[c, :] for c in range(4)]
    Bc = [par_v[4 + c, :] for c in range(4)]
    tv = par_v[8, :]
    lov = par_v[9, :]
    iota16 = lax.broadcasted_iota(jnp.int32, (16,), 0)
    semg = (semg0, semg1)
    seme = (seme0, seme1)

    def _zz(i, carry):
        for c in range(8):
            zbuf[i, pl.ds(c * 16, 16)] = jnp.zeros((16,), jnp.float32)
        return carry
    lax.fori_loop(0, 32, _zz, 0)

    def _start_chunk(m, k):
        pltpu.sync_copy(sd_hbm.at[m], sd_i.at[k])
        pltpu.async_copy(h_hbm.at[sd_i.at[k, 0]], rows.at[k], semg[k])
        pltpu.async_copy(ea_hbm.at[pl.ds(m * CH, CH)], eab.at[k], seme[k])

    def _wait_chunk(m, k):
        pltpu.make_async_copy(h_hbm.at[sd_i.at[k, 0]], rows.at[k],
                              semg[k]).wait()
        pltpu.make_async_copy(ea_hbm.at[pl.ds(m * CH, CH)], eab.at[k],
                              seme[k]).wait()

    for kq in range(NPART // NC):
        q = cid * (NPART // NC) + kq
        for off in range(0, RPW, 32):
            pltpu.sync_copy(zbuf.at[...],
                            acc.at[pl.ds(sid * RPW + off, 32)])
        plsc.subcore_barrier()

        ws = meta_v[pl.ds(q * 32 + sid, 16)][0].astype(jnp.int32)
        we = meta_v[pl.ds(q * 32 + 16 + sid, 16)][0].astype(jnp.int32)
        qb = q * NQ
        m0 = lax.div(ws, CH)
        nch = lax.div(we - ws + CH - 1, CH)
        n2 = lax.div(nch + 1, 2)

        _start_chunk(m0, 0)

        def _pair(j2, carry):
            for k in range(2):
                mk = m0 + 2 * j2 + k
                _start_chunk(mk + 1, 1 - k)
                _wait_chunk(mk, k)
                e0 = mk * CH
                for v in range(8):
                    d16 = sd_i[k, 1, pl.ds(v * 16, 16)]
                    dl = d16 - qb
                    eidx = (e0 + v * 16) + iota16
                    ok = (dl >= 0) & (dl < NQ) & (eidx < we)
                    idx_i[k, pl.ds(v * 16, 16)] = jnp.where(ok, dl, NQ)

                def _vals(i4, carry2):
                    for u in range(4):
                        i = i4 * 4 + u
                        for c in range(4):
                            r = rows[k, i, pl.ds(c * 16, 16)]
                            a = eab[k, i, pl.ds(c * 16, 16)]
                            gv = r * Ac[c] + Bc[c]
                            g = jnp.maximum(gv, lov * gv)
                            m = jnp.maximum(g + a, 0.0) + EPS_GEN
                            ev = jnp.exp(tv * m)
                            vb[k, i, pl.ds(c * 16, 16)] = ev
                            vb[k, i, pl.ds(64 + c * 16, 16)] = m * ev
                    return carry2
                lax.fori_loop(0, 32, _vals, 0)
                pltpu.sync_copy(vb.at[k], acc.at[idx_i.at[k]], add=True)
            return carry
        lax.fori_loop(0, n2, _pair, 0)
        _wait_chunk(m0 + 2 * n2, 0)
        plsc.subcore_barrier()

        for off in range(0, RPW, 128):
            sz = min(128, RPW - off)
            pltpu.sync_copy(
                acc.at[pl.ds(sid * RPW + off, sz)],
                acc_hbm.at[q, pl.ds(sid * RPW + off, sz)])
        plsc.subcore_barrier()


_sc_edge = pl.kernel(
    _sc_edge_body,
    out_type=jax.ShapeDtypeStruct((NPART, NQP, 128), jnp.float32),
    mesh=plsc.VectorSubcoreMesh(core_axis_name="c", subcore_axis_name="s"),
    scratch_types=[
        pltpu.VMEM((672,), jnp.float32),
        pltpu.VMEM((16, 16), jnp.float32),
        pltpu.VMEM((2, 2, 128), jnp.int32),
        pltpu.VMEM((2, 128), jnp.int32),
        pltpu.VMEM((2, 128, 128), jnp.float32),
        pltpu.VMEM((2, 128, 64), jnp.float32),
        pltpu.VMEM((2, 128, 128), jnp.float32),
        pltpu.VMEM((32, 128), jnp.float32),
        pltpu.VMEM_SHARED((NQP, 128), jnp.float32),
        pltpu.SemaphoreType.DMA,
        pltpu.SemaphoreType.DMA,
        pltpu.SemaphoreType.DMA,
        pltpu.SemaphoreType.DMA,
    ],
)


def _rowmask(i, n_rows, total):
    rows = i * BLK + lax.broadcasted_iota(jnp.int32, (n_rows, 1), 0)
    return rows < total


def _enc_atom_body(x0, x1, x2, x3, t0, t1, t2, t3, h0_ref):
    xs = (x0, x1, x2, x3)
    ts = (t0, t1, t2, t3)
    acc = jnp.zeros((BLK, EMB), jnp.float32)
    lanes = lax.broadcasted_iota(jnp.int32, (BLK, 128), 1)
    for k in range(4):
        xv = xs[k][0, 0, :]
        oh = (lanes == xv[:, None]).astype(jnp.float32)
        acc = acc + jnp.dot(oh, ts[k][...],
                            preferred_element_type=jnp.float32, precision=lax.Precision.HIGHEST)
    h0_ref[...] = jnp.concatenate(
        [acc, jnp.zeros((BLK, 128 - EMB), jnp.float32)], axis=1)


_enc_atom = pl.pallas_call(
    _enc_atom_body,
    grid=(NBLK,),
    in_specs=[pl.BlockSpec((1, 1, BLK), lambda i: (i, 0, 0))] * 4
    + [pl.BlockSpec((128, EMB), lambda i: (0, 0))] * 4,
    out_specs=pl.BlockSpec((BLK, 128), lambda i: (i, 0)),
    out_shape=jax.ShapeDtypeStruct((N_NODES, 128), jnp.float32),
)


def _enc_bond_body(a0, a1, a2, t0, t1, t2, ea_ref):
    As = (a0, a1, a2)
    ts = (t0, t1, t2)
    acc = jnp.zeros((BLK, EMB), jnp.float32)
    lanes = lax.broadcasted_iota(jnp.int32, (BLK, 16), 1)
    for k in range(3):
        xv = As[k][0, 0, :]
        oh = (lanes == xv[:, None]).astype(jnp.float32)
        acc = acc + jnp.dot(oh, ts[k][...],
                            preferred_element_type=jnp.float32, precision=lax.Precision.HIGHEST)
    ea_ref[...] = acc


_enc_bond = pl.pallas_call(
    _enc_bond_body,
    grid=(EBLK,),
    in_specs=[pl.BlockSpec((1, 1, BLK), lambda i: (i, 0, 0))] * 3
    + [pl.BlockSpec((16, EMB), lambda i: (0, 0))] * 3,
    out_specs=pl.BlockSpec((BLK, EMB), lambda i: (i, 0)),
    out_shape=jax.ShapeDtypeStruct((EA_ROWS, EMB), jnp.float32),
)


def _mm1_body(acc_ref, h_ref, W1_ref, b1_ref, A_ref, B_ref, lo_ref,
              z_ref, ps_ref, pss_ref):
    i = pl.program_id(0)
    accv = acc_ref[...]
    h = h_ref[:, :EMB]
    v = h * A_ref[...] + B_ref[...]
    g = jnp.maximum(v, lo_ref[...] * v)
    d = accv[:, :EMB]
    s = accv[:, EMB:]
    out = s / (d + 1e-16) + g
    z = jnp.dot(out, W1_ref[...], preferred_element_type=jnp.float32)
    z = z + b1_ref[...]
    z = jnp.where(_rowmask(i, BLK, N_NODES), z, 0.0)
    z_ref[...] = z
    ps_ref[...] = jnp.sum(z, axis=0, keepdims=True)[None]
    pss_ref[...] = jnp.sum(z * z, axis=0, keepdims=True)[None]


_mm1 = pl.pallas_call(
    _mm1_body,
    grid=(NBLK,),
    in_specs=[
        pl.BlockSpec((BLK, 128), lambda i: (i, 0)),
        pl.BlockSpec((BLK, 128), lambda i: (i, 0)),
        pl.BlockSpec((EMB, HID), lambda i: (0, 0)),
        pl.BlockSpec((1, HID), lambda i: (0, 0)),
        pl.BlockSpec((1, EMB), lambda i: (0, 0)),
        pl.BlockSpec((1, EMB), lambda i: (0, 0)),
        pl.BlockSpec((1, EMB), lambda i: (0, 0)),
    ],
    out_specs=[
        pl.BlockSpec((BLK, HID), lambda i: (i, 0)),
        pl.BlockSpec((1, 1, HID), lambda i: (i, 0, 0)),
        pl.BlockSpec((1, 1, HID), lambda i: (i, 0, 0)),
    ],
    out_shape=[
        jax.ShapeDtypeStruct((N_NODES, HID), jnp.float32),
        jax.ShapeDtypeStruct((NBLK, 1, HID), jnp.float32),
        jax.ShapeDtypeStruct((NBLK, 1, HID), jnp.float32),
    ],
)


def _mm2_body(z_ref, h_ref, zA_ref, zB_ref, W2_ref, b2_ref, rm_ref,
              h2_ref, hs_ref, hss_ref):
    i = pl.program_id(0)
    z = z_ref[...]
    zn = jnp.maximum(z * zA_ref[...] + zB_ref[...], 0.0)
    y = jnp.dot(zn, W2_ref[...], preferred_element_type=jnp.float32)
    y = y + b2_ref[...]
    h2 = rm_ref[...] * h_ref[:, :EMB] + y
    h2 = jnp.where(_rowmask(i, BLK, N_NODES), h2, 0.0)
    h2_ref[...] = jnp.concatenate(
        [h2, jnp.zeros((BLK, 128 - EMB), jnp.float32)], axis=1)
    hs_ref[...] = jnp.sum(h2, axis=0, keepdims=True)[None]
    hss_ref[...] = jnp.sum(h2 * h2, axis=0, keepdims=True)[None]


_mm2 = pl.pallas_call(
    _mm2_body,
    grid=(NBLK,),
    in_specs=[
        pl.BlockSpec((BLK, HID), lambda i: (i, 0)),
        pl.BlockSpec((BLK, 128), lambda i: (i, 0)),
        pl.BlockSpec((1, HID), lambda i: (0, 0)),
        pl.BlockSpec((1, HID), lambda i: (0, 0)),
        pl.BlockSpec((HID, EMB), lambda i: (0, 0)),
        pl.BlockSpec((1, EMB), lambda i: (0, 0)),
        pl.BlockSpec((1, EMB), lambda i: (0, 0)),
    ],
    out_specs=[
        pl.BlockSpec((BLK, 128), lambda i: (i, 0)),
        pl.BlockSpec((1, 1, EMB), lambda i: (i, 0, 0)),
        pl.BlockSpec((1, 1, EMB), lambda i: (i, 0, 0)),
    ],
    out_shape=[
        jax.ShapeDtypeStruct((N_NODES, 128), jnp.float32),
        jax.ShapeDtypeStruct((NBLK, 1, EMB), jnp.float32),
        jax.ShapeDtypeStruct((NBLK, 1, EMB), jnp.float32),
    ],
)


def _pool_body(h_ref, b_ref, A_ref, B_ref, Wp_ref, bp_ref, o_ref,
               pooled, cnt):
    i = pl.program_id(0)

    @pl.when(i == 0)
    def _():
        pooled[...] = jnp.zeros_like(pooled)
        cnt[...] = jnp.zeros_like(cnt)

    hn = h_ref[:, :EMB] * A_ref[...] + B_ref[...]
    hn = jnp.where(_rowmask(i, BLK, N_NODES), hn, 0.0)
    bv = b_ref[0, 0, :]
    gi = lax.broadcasted_iota(jnp.int32, (N_GRAPHS, BLK), 0)
    P = (gi == bv[None, :]).astype(jnp.float32)
    pooled[...] += jnp.dot(P, hn, preferred_element_type=jnp.float32, precision=lax.Precision.HIGHEST)
    cnt[...] += jnp.dot(P, jnp.ones((BLK, EMB), jnp.float32),
                        preferred_element_type=jnp.float32, precision=lax.Precision.HIGHEST)

    @pl.when(i == NBLK - 1)
    def _():
        gm = pooled[...] / jnp.maximum(cnt[...], 1.0)
        o_ref[...] = jnp.dot(gm, Wp_ref[...],
                             preferred_element_type=jnp.float32, precision=lax.Precision.HIGHEST) + bp_ref[...]


_pool = pl.pallas_call(
    _pool_body,
    grid=(NBLK,),
    in_specs=[
        pl.BlockSpec((BLK, 128), lambda i: (i, 0)),
        pl.BlockSpec((1, 1, BLK), lambda i: (i, 0, 0)),
        pl.BlockSpec((1, EMB), lambda i: (0, 0)),
        pl.BlockSpec((1, EMB), lambda i: (0, 0)),
        pl.BlockSpec((EMB, 128), lambda i: (0, 0)),
        pl.BlockSpec((1, 128), lambda i: (0, 0)),
    ],
    out_specs=pl.BlockSpec((N_GRAPHS, 128), lambda i: (0, 0)),
    out_shape=jax.ShapeDtypeStruct((N_GRAPHS, 128), jnp.float32),
    scratch_shapes=[
        pltpu.VMEM((N_GRAPHS, EMB), jnp.float32),
        pltpu.VMEM((N_GRAPHS, EMB), jnp.float32),
    ],
)


def _pad_to(a, n, fill):
    return jnp.concatenate(
        [a, jnp.full((n - a.shape[0],) + a.shape[1:], fill, a.dtype)])


def _blocks3d(col, nblk, fill):
    return _pad_to(col, nblk * BLK, fill).reshape(nblk, BLK)[:, None, :]


def kernel(x, edge_index, edge_attr, batch, atom_emb, bond_emb, t, W1, b1,
           bn_g, bn_b, W2, b2, ln_g, ln_b, Wp, bp):
    f32 = jnp.float32
    src = edge_index[0].astype(jnp.int32)
    dst = edge_index[1].astype(jnp.int32)
    perm = jnp.argsort(dst)
    dsts = dst[perm]
    srcs = src[perm]
    attr_s = edge_attr[perm].astype(jnp.int32)

    srcs_p = _pad_to(srcs, EPAD, 0)
    dsts_p = _pad_to(dsts, EPAD, 1 << 29)
    sd = jnp.stack([srcs_p.reshape(EPAD // CH, CH),
                    dsts_p.reshape(EPAD // CH, CH)], axis=1)

    qb = jnp.searchsorted(
        dsts, jnp.arange(0, 50001, NQ, dtype=jnp.int32)).astype(jnp.int32)
    w = jnp.arange(NS, dtype=jnp.int32)
    lo_q = qb[:NPART][:, None]
    hi_q = qb[1:][:, None]
    starts = (lo_q + (w[None, :] * (hi_q - lo_q)) // NS) & ~127
    ends = jnp.concatenate([starts[:, 1:], hi_q], axis=1)
    meta = jnp.concatenate(
        [jnp.stack([starts, ends], axis=1).astype(jnp.float32).reshape(
            NPART * 32),
         jnp.zeros((32,), jnp.float32)])
    meta = _pad_to(meta, 672, 0.0)

    xT = [_blocks3d(x[:, k].astype(jnp.int32), NBLK, 0) for k in range(4)]
    h = _enc_atom(*xT, atom_emb[0], atom_emb[1], atom_emb[2], atom_emb[3])
    aT = [_blocks3d(attr_s[:, k], EBLK, 0) for k in range(3)]
    ea_s = _enc_bond(*aT, bond_emb[0], bond_emb[1], bond_emb[2])

    ones64 = jnp.ones((EMB,), f32)
    zeros64 = jnp.zeros((EMB,), f32)
    A, B, lo = ones64, zeros64, 1.0

    for i in range(N_LAYERS):
        par = jnp.zeros((16, 16), f32)
        par = par.at[0:4, :].set(A.reshape(4, 16))
        par = par.at[4:8, :].set(B.reshape(4, 16))
        par = par.at[8, :].set(t[i])
        par = par.at[9, :].set(lo)
        acc4 = _sc_edge(h, ea_s, sd, meta, par)
        acc = acc4[:, :NQ, :].reshape(N_NODES, 128)
        z, ps, pss = _mm1(acc, h, W1[i], b1[i].reshape(1, -1),
                          A.reshape(1, -1), B.reshape(1, -1),
                          jnp.full((1, EMB), lo, f32))
        mu = jnp.sum(ps, (0, 1)) / N_NODES
        var = jnp.sum(pss, (0, 1)) / N_NODES - mu * mu
        zA = bn_g[i] * lax.rsqrt(var + EPS_BN)
        zB = bn_b[i] - mu * zA
        rm = jnp.full((1, EMB), 0.0 if i == 0 else 1.0, f32)
        h, hs, hss = _mm2(z, h, zA.reshape(1, -1), zB.reshape(1, -1),
                          W2[i], b2[i].reshape(1, -1), rm)
        muh = jnp.sum(hs, (0, 1)) / N_NODES
        varh = jnp.sum(hss, (0, 1)) / N_NODES - muh * muh
        j = i + 1 if i + 1 < N_LAYERS else 0
        A = ln_g[j] * lax.rsqrt(varh + EPS_BN)
        B = ln_b[j] - muh * A
        lo = 0.0

    b3d = _blocks3d(batch.astype(jnp.int32), NBLK, 1 << 29)
    Wp_pad = jnp.zeros((EMB, 128), f32).at[:, :Wp.shape[1]].set(Wp)
    bp_pad = jnp.zeros((1, 128), f32).at[0, :bp.shape[0]].set(bp)
    outp = _pool(h, b3d, A.reshape(1, -1), B.reshape(1, -1), Wp_pad, bp_pad)
    return outp[:, :Wp.shape[1]]


# speedup vs baseline: 9.4927x; 1.0141x over previous
import functools

import jax
import jax.numpy as jnp
from jax import lax
from jax.experimental import pallas as pl
from jax.experimental.pallas import tpu as pltpu
from jax.experimental.pallas import tpu_sc as plsc

N_NODES = 50000
N_EDGES = 800000
EMB = 64
HID = 128
N_LAYERS = 7
N_GRAPHS = 128
EPS_GEN = 1e-7
EPS_BN = 1e-5

NC = 2
NS = 16
NQ = 2500
NPART = 20
NQP = 2560
RPW = NQP // NS
CH = 128
EPAD = 800512
BLK = 512
NBLK = 98
EBLK = 1564
EA_ROWS = EBLK * BLK


def _lane_extract(vec, lane):
    lanes = lax.broadcasted_iota(jnp.int32, (16,), 0)
    s = jnp.sum(jnp.where(lanes == lane, vec, 0.0))
    return s.astype(jnp.int32)


def _sc_edge_body(h_hbm, ea_hbm, sd_hbm, meta_hbm, par_hbm,
                  acc_hbm, meta_v, par_v, sd_i, idx_i, rows, eab, vb, zbuf,
                  acc, semg0, semg1, seme0, seme1):
    cid = lax.axis_index("c")
    sid = lax.axis_index("s")
    pltpu.sync_copy(meta_hbm, meta_v)
    pltpu.sync_copy(par_hbm, par_v)
    Ac = [par_v[c, :] for c in range(4)]
    Bc = [par_v[4 + c, :] for c in range(4)]
    tv = par_v[8, :]
    lov = par_v[9, :]
    iota16 = lax.broadcasted_iota(jnp.int32, (16,), 0)
    semg = (semg0, semg1)
    seme = (seme0, seme1)

    def _zz(i, carry):
        for c in range(8):
            zbuf[i, pl.ds(c * 16, 16)] = jnp.zeros((16,), jnp.float32)
        return carry
    lax.fori_loop(0, 32, _zz, 0)

    def _start_chunk(m, k):
        pltpu.sync_copy(sd_hbm.at[m], sd_i.at[k])
        pltpu.async_copy(h_hbm.at[sd_i.at[k, 0]], rows.at[k], semg[k])
        pltpu.async_copy(ea_hbm.at[pl.ds(m * CH, CH)], eab.at[k], seme[k])

    def _wait_chunk(m, k):
        pltpu.make_async_copy(h_hbm.at[sd_i.at[k, 0]], rows.at[k],
                              semg[k]).wait()
        pltpu.make_async_copy(ea_hbm.at[pl.ds(m * CH, CH)], eab.at[k],
                              seme[k]).wait()

    for kq in range(NPART // NC):
        q = cid * (NPART // NC) + kq
        for off in range(0, RPW, 32):
            pltpu.sync_copy(zbuf.at[...],
                            acc.at[pl.ds(sid * RPW + off, 32)])
        plsc.subcore_barrier()

        ws = meta_v[pl.ds(q * 32 + sid, 16)][0].astype(jnp.int32)
        we = meta_v[pl.ds(q * 32 + 16 + sid, 16)][0].astype(jnp.int32)
        qb = q * NQ
        m0 = lax.div(ws, CH)
        nch = lax.div(we - ws + CH - 1, CH)
        n2 = lax.div(nch + 1, 2)

        _start_chunk(m0, 0)

        def _pair(j2, carry):
            for k in range(2):
                mk = m0 + 2 * j2 + k
                _start_chunk(mk + 1, 1 - k)
                _wait_chunk(mk, k)
                e0 = mk * CH
                for v in range(8):
                    d16 = sd_i[k, 1, pl.ds(v * 16, 16)]
                    dl = d16 - qb
                    eidx = (e0 + v * 16) + iota16
                    ok = (dl >= 0) & (dl < NQ) & (eidx < we)
                    idx_i[k, pl.ds(v * 16, 16)] = jnp.where(ok, dl, NQ)

                def _vals(i4, carry2):
                    for u in range(4):
                        i = i4 * 4 + u
                        for c in range(4):
                            r = rows[k, i, pl.ds(c * 16, 16)]
                            a = eab[k, i, pl.ds(c * 16, 16)]
                            gv = r * Ac[c] + Bc[c]
                            g = jnp.maximum(gv, lov * gv)
                            m = jnp.maximum(g + a, 0.0) + EPS_GEN
                            ev = jnp.exp(tv * m)
                            vb[k, i, pl.ds(c * 16, 16)] = ev
                            vb[k, i, pl.ds(64 + c * 16, 16)] = m * ev
                    return carry2
                lax.fori_loop(0, 32, _vals, 0)
                pltpu.sync_copy(vb.at[k], acc.at[idx_i.at[k]], add=True)
            return carry
        lax.fori_loop(0, n2, _pair, 0)
        _wait_chunk(m0 + 2 * n2, 0)
        plsc.subcore_barrier()

        for off in range(0, RPW, 128):
            sz = min(128, RPW - off)
            pltpu.sync_copy(
                acc.at[pl.ds(sid * RPW + off, sz)],
                acc_hbm.at[q, pl.ds(sid * RPW + off, sz)])
        plsc.subcore_barrier()


_sc_edge = pl.kernel(
    _sc_edge_body,
    out_type=jax.ShapeDtypeStruct((NPART, NQP, 128), jnp.float32),
    mesh=plsc.VectorSubcoreMesh(core_axis_name="c", subcore_axis_name="s"),
    scratch_types=[
        pltpu.VMEM((672,), jnp.float32),
        pltpu.VMEM((16, 16), jnp.float32),
        pltpu.VMEM((2, 2, 128), jnp.int32),
        pltpu.VMEM((2, 128), jnp.int32),
        pltpu.VMEM((2, 128, 128), jnp.float32),
        pltpu.VMEM((2, 128, 64), jnp.float32),
        pltpu.VMEM((2, 128, 128), jnp.float32),
        pltpu.VMEM((32, 128), jnp.float32),
        pltpu.VMEM_SHARED((NQP, 128), jnp.float32),
        pltpu.SemaphoreType.DMA,
        pltpu.SemaphoreType.DMA,
        pltpu.SemaphoreType.DMA,
        pltpu.SemaphoreType.DMA,
    ],
)


def _rowmask(i, n_rows, total):
    rows = i * BLK + lax.broadcasted_iota(jnp.int32, (n_rows, 1), 0)
    return rows < total


def _enc_atom_body(x0, x1, x2, x3, t0, t1, t2, t3, h0_ref):
    xs = (x0, x1, x2, x3)
    ts = (t0, t1, t2, t3)
    acc = jnp.zeros((BLK, EMB), jnp.float32)
    lanes = lax.broadcasted_iota(jnp.int32, (BLK, 128), 1)
    for k in range(4):
        xv = xs[k][0, 0, :]
        oh = (lanes == xv[:, None]).astype(jnp.float32)
        acc = acc + jnp.dot(oh, ts[k][...],
                            preferred_element_type=jnp.float32, precision=lax.Precision.HIGHEST)
    h0_ref[...] = jnp.concatenate(
        [acc, jnp.zeros((BLK, 128 - EMB), jnp.float32)], axis=1)


_enc_atom = pl.pallas_call(
    _enc_atom_body,
    grid=(NBLK,),
    in_specs=[pl.BlockSpec((1, 1, BLK), lambda i: (i, 0, 0))] * 4
    + [pl.BlockSpec((128, EMB), lambda i: (0, 0))] * 4,
    out_specs=pl.BlockSpec((BLK, 128), lambda i: (i, 0)),
    out_shape=jax.ShapeDtypeStruct((N_NODES, 128), jnp.float32),
)


def _enc_bond_body(a0, a1, a2, t0, t1, t2, ea_ref):
    As = (a0, a1, a2)
    ts = (t0, t1, t2)
    acc = jnp.zeros((BLK, EMB), jnp.float32)
    lanes = lax.broadcasted_iota(jnp.int32, (BLK, 16), 1)
    for k in range(3):
        xv = As[k][0, 0, :]
        oh = (lanes == xv[:, None]).astype(jnp.float32)
        acc = acc + jnp.dot(oh, ts[k][...],
                            preferred_element_type=jnp.float32, precision=lax.Precision.HIGHEST)
    ea_ref[...] = acc


_enc_bond = pl.pallas_call(
    _enc_bond_body,
    grid=(EBLK,),
    in_specs=[pl.BlockSpec((1, 1, BLK), lambda i: (i, 0, 0))] * 3
    + [pl.BlockSpec((16, EMB), lambda i: (0, 0))] * 3,
    out_specs=pl.BlockSpec((BLK, EMB), lambda i: (i, 0)),
    out_shape=jax.ShapeDtypeStruct((EA_ROWS, EMB), jnp.float32),
)


def _layer_body(acc_ref, h_ref, W1_ref, b1_ref, A_ref, B_ref, lo_ref,
                bng_ref, bnb_ref, W2_ref, b2_ref, rm_ref,
                h2_ref, hst_ref, zbuf, zstat, zab, hstat):
    i = pl.program_id(0)

    @pl.when(i == 0)
    def _():
        zstat[...] = jnp.zeros_like(zstat)
        hstat[...] = jnp.zeros_like(hstat)

    @pl.when(i < NBLK)
    def _():
        accv = acc_ref[...]
        h = h_ref[:, :EMB]
        v = h * A_ref[...] + B_ref[...]
        g = jnp.maximum(v, lo_ref[...] * v)
        out = accv[:, EMB:] / (accv[:, :EMB] + 1e-16) + g
        z = jnp.dot(out, W1_ref[...], preferred_element_type=jnp.float32)
        z = z + b1_ref[...]
        z = jnp.where(_rowmask(i, BLK, N_NODES), z, 0.0)
        zbuf[pl.ds(i * BLK, BLK), :] = z
        zstat[0:1, :] += jnp.sum(z, axis=0, keepdims=True)
        zstat[1:2, :] += jnp.sum(z * z, axis=0, keepdims=True)

    @pl.when(i == NBLK)
    def _():
        mu = zstat[0:1, :] * (1.0 / N_NODES)
        var = zstat[1:2, :] * (1.0 / N_NODES) - mu * mu
        zA = bng_ref[...] * lax.rsqrt(var + EPS_BN)
        zab[0:1, :] = zA
        zab[1:2, :] = bnb_ref[...] - mu * zA

    @pl.when(i > NBLK)
    def _():
        j = i - NBLK - 1
        z = zbuf[pl.ds(j * BLK, BLK), :]
        zn = jnp.maximum(z * zab[0:1, :] + zab[1:2, :], 0.0)
        y = jnp.dot(zn, W2_ref[...], preferred_element_type=jnp.float32)
        y = y + b2_ref[...]
        h2 = rm_ref[...] * h_ref[:, :EMB] + y
        h2 = jnp.where(_rowmask(j, BLK, N_NODES), h2, 0.0)
        h2_ref[...] = jnp.concatenate(
            [h2, jnp.zeros((BLK, 128 - EMB), jnp.float32)], axis=1)
        hstat[0:1, :] += jnp.sum(h2, axis=0, keepdims=True)
        hstat[1:2, :] += jnp.sum(h2 * h2, axis=0, keepdims=True)

    @pl.when(i == 2 * NBLK)
    def _():
        hst_ref[...] = hstat[...]


def _c0(i):
    return (jnp.where(i < NBLK, i, 0), 0)


def _chb(i):
    return (jnp.where(i < NBLK, i, jnp.maximum(i - NBLK - 1, 0)), 0)


def _cout(i):
    return (jnp.maximum(i - NBLK - 1, 0), 0)


_layer = pl.pallas_call(
    _layer_body,
    grid=(2 * NBLK + 1,),
    in_specs=[
        pl.BlockSpec((BLK, 128), _c0),
        pl.BlockSpec((BLK, 128), _chb),
        pl.BlockSpec((EMB, HID), lambda i: (0, 0)),
        pl.BlockSpec((1, HID), lambda i: (0, 0)),
        pl.BlockSpec((1, EMB), lambda i: (0, 0)),
        pl.BlockSpec((1, EMB), lambda i: (0, 0)),
        pl.BlockSpec((1, EMB), lambda i: (0, 0)),
        pl.BlockSpec((1, HID), lambda i: (0, 0)),
        pl.BlockSpec((1, HID), lambda i: (0, 0)),
        pl.BlockSpec((HID, EMB), lambda i: (0, 0)),
        pl.BlockSpec((1, EMB), lambda i: (0, 0)),
        pl.BlockSpec((1, EMB), lambda i: (0, 0)),
    ],
    out_specs=[
        pl.BlockSpec((BLK, 128), _cout),
        pl.BlockSpec((2, EMB), lambda i: (0, 0)),
    ],
    out_shape=[
        jax.ShapeDtypeStruct((N_NODES, 128), jnp.float32),
        jax.ShapeDtypeStruct((2, EMB), jnp.float32),
    ],
    scratch_shapes=[
        pltpu.VMEM((NBLK * BLK, HID), jnp.float32),
        pltpu.VMEM((2, HID), jnp.float32),
        pltpu.VMEM((2, HID), jnp.float32),
        pltpu.VMEM((2, EMB), jnp.float32),
    ],
)


def _pool_body(h_ref, b_ref, A_ref, B_ref, Wp_ref, bp_ref, o_ref,
               pooled, cnt):
    i = pl.program_id(0)

    @pl.when(i == 0)
    def _():
        pooled[...] = jnp.zeros_like(pooled)
        cnt[...] = jnp.zeros_like(cnt)

    hn = h_ref[:, :EMB] * A_ref[...] + B_ref[...]
    hn = jnp.where(_rowmask(i, BLK, N_NODES), hn, 0.0)
    bv = b_ref[0, 0, :]
    gi = lax.broadcasted_iota(jnp.int32, (N_GRAPHS, BLK), 0)
    P = (gi == bv[None, :]).astype(jnp.float32)
    pooled[...] += jnp.dot(P, hn, preferred_element_type=jnp.float32, precision=lax.Precision.HIGHEST)
    cnt[...] += jnp.dot(P, jnp.ones((BLK, EMB), jnp.float32),
                        preferred_element_type=jnp.float32, precision=lax.Precision.HIGHEST)

    @pl.when(i == NBLK - 1)
    def _():
        gm = pooled[...] / jnp.maximum(cnt[...], 1.0)
        o_ref[...] = jnp.dot(gm, Wp_ref[...],
                             preferred_element_type=jnp.float32, precision=lax.Precision.HIGHEST) + bp_ref[...]


_pool = pl.pallas_call(
    _pool_body,
    grid=(NBLK,),
    in_specs=[
        pl.BlockSpec((BLK, 128), lambda i: (i, 0)),
        pl.BlockSpec((1, 1, BLK), lambda i: (i, 0, 0)),
        pl.BlockSpec((1, EMB), lambda i: (0, 0)),
        pl.BlockSpec((1, EMB), lambda i: (0, 0)),
        pl.BlockSpec((EMB, 128), lambda i: (0, 0)),
        pl.BlockSpec((1, 128), lambda i: (0, 0)),
    ],
    out_specs=pl.BlockSpec((N_GRAPHS, 128), lambda i: (0, 0)),
    out_shape=jax.ShapeDtypeStruct((N_GRAPHS, 128), jnp.float32),
    scratch_shapes=[
        pltpu.VMEM((N_GRAPHS, EMB), jnp.float32),
        pltpu.VMEM((N_GRAPHS, EMB), jnp.float32),
    ],
)


def _pad_to(a, n, fill):
    return jnp.concatenate(
        [a, jnp.full((n - a.shape[0],) + a.shape[1:], fill, a.dtype)])


def _blocks3d(col, nblk, fill):
    return _pad_to(col, nblk * BLK, fill).reshape(nblk, BLK)[:, None, :]


def kernel(x, edge_index, edge_attr, batch, atom_emb, bond_emb, t, W1, b1,
           bn_g, bn_b, W2, b2, ln_g, ln_b, Wp, bp):
    f32 = jnp.float32
    src = edge_index[0].astype(jnp.int32)
    dst = edge_index[1].astype(jnp.int32)
    perm = jnp.argsort(dst)
    dsts = dst[perm]
    srcs = src[perm]
    attr_s = edge_attr[perm].astype(jnp.int32)

    srcs_p = _pad_to(srcs, EPAD, 0)
    dsts_p = _pad_to(dsts, EPAD, 1 << 29)
    sd = jnp.stack([srcs_p.reshape(EPAD // CH, CH),
                    dsts_p.reshape(EPAD // CH, CH)], axis=1)

    qb = jnp.searchsorted(
        dsts, jnp.arange(0, 50001, NQ, dtype=jnp.int32)).astype(jnp.int32)
    w = jnp.arange(NS, dtype=jnp.int32)
    lo_q = qb[:NPART][:, None]
    hi_q = qb[1:][:, None]
    starts = (lo_q + (w[None, :] * (hi_q - lo_q)) // NS) & ~127
    ends = jnp.concatenate([starts[:, 1:], hi_q], axis=1)
    meta = jnp.concatenate(
        [jnp.stack([starts, ends], axis=1).astype(jnp.float32).reshape(
            NPART * 32),
         jnp.zeros((32,), jnp.float32)])
    meta = _pad_to(meta, 672, 0.0)

    xT = [_blocks3d(x[:, k].astype(jnp.int32), NBLK, 0) for k in range(4)]
    h = _enc_atom(*xT, atom_emb[0], atom_emb[1], atom_emb[2], atom_emb[3])
    aT = [_blocks3d(attr_s[:, k], EBLK, 0) for k in range(3)]
    ea_s = _enc_bond(*aT, bond_emb[0], bond_emb[1], bond_emb[2])

    ones64 = jnp.ones((EMB,), f32)
    zeros64 = jnp.zeros((EMB,), f32)
    A, B, lo = ones64, zeros64, 1.0

    for i in range(N_LAYERS):
        par = jnp.zeros((16, 16), f32)
        par = par.at[0:4, :].set(A.reshape(4, 16))
        par = par.at[4:8, :].set(B.reshape(4, 16))
        par = par.at[8, :].set(t[i])
        par = par.at[9, :].set(lo)
        acc4 = _sc_edge(h, ea_s, sd, meta, par)
        acc = acc4[:, :NQ, :].reshape(N_NODES, 128)
        rm = jnp.full((1, EMB), 0.0 if i == 0 else 1.0, jnp.float32)
        h, hst = _layer(acc, h, W1[i], b1[i].reshape(1, -1),
                        A.reshape(1, -1), B.reshape(1, -1),
                        jnp.full((1, EMB), lo, f32),
                        bn_g[i].reshape(1, -1), bn_b[i].reshape(1, -1),
                        W2[i], b2[i].reshape(1, -1), rm)
        muh = hst[0] / N_NODES
        varh = hst[1] / N_NODES - muh * muh
        j = i + 1 if i + 1 < N_LAYERS else 0
        A = ln_g[j] * lax.rsqrt(varh + EPS_BN)
        B = ln_b[j] - muh * A
        lo = 0.0

    b3d = _blocks3d(batch.astype(jnp.int32), NBLK, 1 << 29)
    Wp_pad = jnp.zeros((EMB, 128), f32).at[:, :Wp.shape[1]].set(Wp)
    bp_pad = jnp.zeros((1, 128), f32).at[0, :bp.shape[0]].set(bp)
    outp = _pool(h, b3d, A.reshape(1, -1), B.reshape(1, -1), Wp_pad, bp_pad)
    return outp[:, :Wp.shape[1]]


# speedup vs baseline: 10.6195x; 1.1187x over previous
import functools

import jax
import jax.numpy as jnp
from jax import lax
from jax.experimental import pallas as pl
from jax.experimental.pallas import tpu as pltpu
from jax.experimental.pallas import tpu_sc as plsc

N_NODES = 50000
N_EDGES = 800000
EMB = 64
HID = 128
N_LAYERS = 7
N_GRAPHS = 128
EPS_GEN = 1e-7
EPS_BN = 1e-5

NC = 2
NS = 16
NQ = 2500
NPART = 20
NQP = 2560
RPW = NQP // NS
CH = 128
EPAD = 800512
BLK = 2048
NBLK = 25
EBLK = 391
EA_ROWS = EBLK * BLK


def _lane_extract(vec, lane):
    lanes = lax.broadcasted_iota(jnp.int32, (16,), 0)
    s = jnp.sum(jnp.where(lanes == lane, vec, 0.0))
    return s.astype(jnp.int32)


def _sc_edge_body(h_hbm, ea_hbm, sd_hbm, meta_hbm, par_hbm,
                  acc_hbm, meta_v, par_v, sd_i, idx_i, rows, eab, vb, zbuf,
                  acc, semg0, semg1, seme0, seme1):
    cid = lax.axis_index("c")
    sid = lax.axis_index("s")
    pltpu.sync_copy(meta_hbm, meta_v)
    pltpu.sync_copy(par_hbm, par_v)
    Ac = [par_v[c, :] for c in range(4)]
    Bc = [par_v[4 + c, :] for c in range(4)]
    tv = par_v[8, :]
    lov = par_v[9, :]
    iota16 = lax.broadcasted_iota(jnp.int32, (16,), 0)
    semg = (semg0, semg1)
    seme = (seme0, seme1)

    def _zz(i, carry):
        for c in range(8):
            zbuf[i, pl.ds(c * 16, 16)] = jnp.zeros((16,), jnp.float32)
        return carry
    lax.fori_loop(0, 32, _zz, 0)

    def _start_chunk(m, k):
        pltpu.sync_copy(sd_hbm.at[m], sd_i.at[k])
        pltpu.async_copy(h_hbm.at[sd_i.at[k, 0]], rows.at[k], semg[k])
        pltpu.async_copy(ea_hbm.at[pl.ds(m * CH, CH)], eab.at[k], seme[k])

    def _wait_chunk(m, k):
        pltpu.make_async_copy(h_hbm.at[sd_i.at[k, 0]], rows.at[k],
                              semg[k]).wait()
        pltpu.make_async_copy(ea_hbm.at[pl.ds(m * CH, CH)], eab.at[k],
                              seme[k]).wait()

    for kq in range(NPART // NC):
        q = cid * (NPART // NC) + kq
        for off in range(0, RPW, 32):
            pltpu.sync_copy(zbuf.at[...],
                            acc.at[pl.ds(sid * RPW + off, 32)])
        plsc.subcore_barrier()

        ws = meta_v[pl.ds(q * 32 + sid, 16)][0].astype(jnp.int32)
        we = meta_v[pl.ds(q * 32 + 16 + sid, 16)][0].astype(jnp.int32)
        qb = q * NQ
        m0 = lax.div(ws, CH)
        nch = lax.div(we - ws + CH - 1, CH)
        n2 = lax.div(nch + 1, 2)

        _start_chunk(m0, 0)

        def _pair(j2, carry):
            for k in range(2):
                mk = m0 + 2 * j2 + k
                _start_chunk(mk + 1, 1 - k)
                _wait_chunk(mk, k)
                e0 = mk * CH
                for v in range(8):
                    d16 = sd_i[k, 1, pl.ds(v * 16, 16)]
                    dl = d16 - qb
                    eidx = (e0 + v * 16) + iota16
                    ok = (dl >= 0) & (dl < NQ) & (eidx < we)
                    idx_i[k, pl.ds(v * 16, 16)] = jnp.where(ok, dl, NQ)

                def _vals(i4, carry2):
                    for u in range(4):
                        i = i4 * 4 + u
                        for c in range(4):
                            r = rows[k, i, pl.ds(c * 16, 16)]
                            a = eab[k, i, pl.ds(c * 16, 16)]
                            gv = r * Ac[c] + Bc[c]
                            g = jnp.maximum(gv, lov * gv)
                            m = jnp.maximum(g + a, 0.0) + EPS_GEN
                            ev = jnp.exp(tv * m)
                            vb[k, i, pl.ds(c * 16, 16)] = ev
                            vb[k, i, pl.ds(64 + c * 16, 16)] = m * ev
                    return carry2
                lax.fori_loop(0, 32, _vals, 0)
                pltpu.sync_copy(vb.at[k], acc.at[idx_i.at[k]], add=True)
            return carry
        lax.fori_loop(0, n2, _pair, 0)
        _wait_chunk(m0 + 2 * n2, 0)
        plsc.subcore_barrier()

        for off in range(0, RPW, 128):
            sz = min(128, RPW - off)
            pltpu.sync_copy(
                acc.at[pl.ds(sid * RPW + off, sz)],
                acc_hbm.at[q, pl.ds(sid * RPW + off, sz)])
        plsc.subcore_barrier()


_sc_edge = pl.kernel(
    _sc_edge_body,
    out_type=jax.ShapeDtypeStruct((NPART, NQP, 128), jnp.float32),
    mesh=plsc.VectorSubcoreMesh(core_axis_name="c", subcore_axis_name="s"),
    scratch_types=[
        pltpu.VMEM((672,), jnp.float32),
        pltpu.VMEM((16, 16), jnp.float32),
        pltpu.VMEM((2, 2, 128), jnp.int32),
        pltpu.VMEM((2, 128), jnp.int32),
        pltpu.VMEM((2, 128, 128), jnp.float32),
        pltpu.VMEM((2, 128, 64), jnp.float32),
        pltpu.VMEM((2, 128, 128), jnp.float32),
        pltpu.VMEM((32, 128), jnp.float32),
        pltpu.VMEM_SHARED((NQP, 128), jnp.float32),
        pltpu.SemaphoreType.DMA,
        pltpu.SemaphoreType.DMA,
        pltpu.SemaphoreType.DMA,
        pltpu.SemaphoreType.DMA,
    ],
)


def _rowmask(i, n_rows, total):
    rows = i * BLK + lax.broadcasted_iota(jnp.int32, (n_rows, 1), 0)
    return rows < total


def _enc_atom_body(x0, x1, x2, x3, t0, t1, t2, t3, h0_ref):
    xs = (x0, x1, x2, x3)
    ts = (t0, t1, t2, t3)
    acc = jnp.zeros((BLK, EMB), jnp.float32)
    lanes = lax.broadcasted_iota(jnp.int32, (BLK, 128), 1)
    for k in range(4):
        xv = xs[k][0, 0, :]
        oh = (lanes == xv[:, None]).astype(jnp.float32)
        acc = acc + jnp.dot(oh, ts[k][...],
                            preferred_element_type=jnp.float32, precision=lax.Precision.HIGHEST)
    h0_ref[...] = jnp.concatenate(
        [acc, jnp.zeros((BLK, 128 - EMB), jnp.float32)], axis=1)


_enc_atom = pl.pallas_call(
    _enc_atom_body,
    grid=(NBLK,),
    in_specs=[pl.BlockSpec((1, 1, BLK), lambda i: (i, 0, 0))] * 4
    + [pl.BlockSpec((128, EMB), lambda i: (0, 0))] * 4,
    out_specs=pl.BlockSpec((BLK, 128), lambda i: (i, 0)),
    out_shape=jax.ShapeDtypeStruct((N_NODES, 128), jnp.float32),
)


def _enc_bond_body(a0, a1, a2, t0, t1, t2, ea_ref):
    As = (a0, a1, a2)
    ts = (t0, t1, t2)
    acc = jnp.zeros((BLK, EMB), jnp.float32)
    lanes = lax.broadcasted_iota(jnp.int32, (BLK, 16), 1)
    for k in range(3):
        xv = As[k][0, 0, :]
        oh = (lanes == xv[:, None]).astype(jnp.float32)
        acc = acc + jnp.dot(oh, ts[k][...],
                            preferred_element_type=jnp.float32, precision=lax.Precision.HIGHEST)
    ea_ref[...] = acc


_enc_bond = pl.pallas_call(
    _enc_bond_body,
    grid=(EBLK,),
    in_specs=[pl.BlockSpec((1, 1, BLK), lambda i: (i, 0, 0))] * 3
    + [pl.BlockSpec((16, EMB), lambda i: (0, 0))] * 3,
    out_specs=pl.BlockSpec((BLK, EMB), lambda i: (i, 0)),
    out_shape=jax.ShapeDtypeStruct((EA_ROWS, EMB), jnp.float32),
)


def _layer_body(acc_ref, h_ref, W1_ref, b1_ref, A_ref, B_ref, lo_ref,
                bng_ref, bnb_ref, W2_ref, b2_ref, rm_ref,
                h2_ref, hst_ref, zbuf, zstat, zab, hstat):
    i = pl.program_id(0)

    @pl.when(i == 0)
    def _():
        zstat[...] = jnp.zeros_like(zstat)
        hstat[...] = jnp.zeros_like(hstat)

    @pl.when(i < NBLK)
    def _():
        accv = acc_ref[...]
        h = h_ref[:, :EMB]
        v = h * A_ref[...] + B_ref[...]
        g = jnp.maximum(v, lo_ref[...] * v)
        out = accv[:, EMB:] / (accv[:, :EMB] + 1e-16) + g
        z = jnp.dot(out, W1_ref[...], preferred_element_type=jnp.float32)
        z = z + b1_ref[...]
        z = jnp.where(_rowmask(i, BLK, N_NODES), z, 0.0)
        zbuf[pl.ds(i * BLK, BLK), :] = z
        zstat[0:1, :] += jnp.sum(z, axis=0, keepdims=True)
        zstat[1:2, :] += jnp.sum(z * z, axis=0, keepdims=True)

    @pl.when(i == NBLK)
    def _():
        mu = zstat[0:1, :] * (1.0 / N_NODES)
        var = zstat[1:2, :] * (1.0 / N_NODES) - mu * mu
        zA = bng_ref[...] * lax.rsqrt(var + EPS_BN)
        zab[0:1, :] = zA
        zab[1:2, :] = bnb_ref[...] - mu * zA

    @pl.when(i > NBLK)
    def _():
        j = i - NBLK - 1
        z = zbuf[pl.ds(j * BLK, BLK), :]
        zn = jnp.maximum(z * zab[0:1, :] + zab[1:2, :], 0.0)
        y = jnp.dot(zn, W2_ref[...], preferred_element_type=jnp.float32)
        y = y + b2_ref[...]
        h2 = rm_ref[...] * h_ref[:, :EMB] + y
        h2 = jnp.where(_rowmask(j, BLK, N_NODES), h2, 0.0)
        h2_ref[...] = jnp.concatenate(
            [h2, jnp.zeros((BLK, 128 - EMB), jnp.float32)], axis=1)
        hstat[0:1, :] += jnp.sum(h2, axis=0, keepdims=True)
        hstat[1:2, :] += jnp.sum(h2 * h2, axis=0, keepdims=True)

    @pl.when(i == 2 * NBLK)
    def _():
        hst_ref[...] = hstat[...]


def _c0(i):
    return (jnp.where(i < NBLK, i, 0), 0)


def _chb(i):
    return (jnp.where(i < NBLK, i, jnp.maximum(i - NBLK - 1, 0)), 0)


def _cout(i):
    return (jnp.maximum(i - NBLK - 1, 0), 0)


_layer = pl.pallas_call(
    _layer_body,
    grid=(2 * NBLK + 1,),
    in_specs=[
        pl.BlockSpec((BLK, 128), _c0),
        pl.BlockSpec((BLK, 128), _chb),
        pl.BlockSpec((EMB, HID), lambda i: (0, 0)),
        pl.BlockSpec((1, HID), lambda i: (0, 0)),
        pl.BlockSpec((1, EMB), lambda i: (0, 0)),
        pl.BlockSpec((1, EMB), lambda i: (0, 0)),
        pl.BlockSpec((1, EMB), lambda i: (0, 0)),
        pl.BlockSpec((1, HID), lambda i: (0, 0)),
        pl.BlockSpec((1, HID), lambda i: (0, 0)),
        pl.BlockSpec((HID, EMB), lambda i: (0, 0)),
        pl.BlockSpec((1, EMB), lambda i: (0, 0)),
        pl.BlockSpec((1, EMB), lambda i: (0, 0)),
    ],
    out_specs=[
        pl.BlockSpec((BLK, 128), _cout),
        pl.BlockSpec((2, EMB), lambda i: (0, 0)),
    ],
    out_shape=[
        jax.ShapeDtypeStruct((N_NODES, 128), jnp.float32),
        jax.ShapeDtypeStruct((2, EMB), jnp.float32),
    ],
    scratch_shapes=[
        pltpu.VMEM((NBLK * BLK, HID), jnp.float32),
        pltpu.VMEM((2, HID), jnp.float32),
        pltpu.VMEM((2, HID), jnp.float32),
        pltpu.VMEM((2, EMB), jnp.float32),
    ],
)


def _pool_body(h_ref, b_ref, A_ref, B_ref, Wp_ref, bp_ref, o_ref,
               pooled, cnt):
    i = pl.program_id(0)

    @pl.when(i == 0)
    def _():
        pooled[...] = jnp.zeros_like(pooled)
        cnt[...] = jnp.zeros_like(cnt)

    hn = h_ref[:, :EMB] * A_ref[...] + B_ref[...]
    hn = jnp.where(_rowmask(i, BLK, N_NODES), hn, 0.0)
    bv = b_ref[0, 0, :]
    gi = lax.broadcasted_iota(jnp.int32, (N_GRAPHS, BLK), 0)
    P = (gi == bv[None, :]).astype(jnp.float32)
    pooled[...] += jnp.dot(P, hn, preferred_element_type=jnp.float32, precision=lax.Precision.HIGHEST)
    cnt[...] += jnp.dot(P, jnp.ones((BLK, EMB), jnp.float32),
                        preferred_element_type=jnp.float32, precision=lax.Precision.HIGHEST)

    @pl.when(i == NBLK - 1)
    def _():
        gm = pooled[...] / jnp.maximum(cnt[...], 1.0)
        o_ref[...] = jnp.dot(gm, Wp_ref[...],
                             preferred_element_type=jnp.float32, precision=lax.Precision.HIGHEST) + bp_ref[...]


_pool = pl.pallas_call(
    _pool_body,
    grid=(NBLK,),
    in_specs=[
        pl.BlockSpec((BLK, 128), lambda i: (i, 0)),
        pl.BlockSpec((1, 1, BLK), lambda i: (i, 0, 0)),
        pl.BlockSpec((1, EMB), lambda i: (0, 0)),
        pl.BlockSpec((1, EMB), lambda i: (0, 0)),
        pl.BlockSpec((EMB, 128), lambda i: (0, 0)),
        pl.BlockSpec((1, 128), lambda i: (0, 0)),
    ],
    out_specs=pl.BlockSpec((N_GRAPHS, 128), lambda i: (0, 0)),
    out_shape=jax.ShapeDtypeStruct((N_GRAPHS, 128), jnp.float32),
    scratch_shapes=[
        pltpu.VMEM((N_GRAPHS, EMB), jnp.float32),
        pltpu.VMEM((N_GRAPHS, EMB), jnp.float32),
    ],
)


def _pad_to(a, n, fill):
    return jnp.concatenate(
        [a, jnp.full((n - a.shape[0],) + a.shape[1:], fill, a.dtype)])


def _blocks3d(col, nblk, fill):
    return _pad_to(col, nblk * BLK, fill).reshape(nblk, BLK)[:, None, :]


def kernel(x, edge_index, edge_attr, batch, atom_emb, bond_emb, t, W1, b1,
           bn_g, bn_b, W2, b2, ln_g, ln_b, Wp, bp):
    f32 = jnp.float32
    src = edge_index[0].astype(jnp.int32)
    dst = edge_index[1].astype(jnp.int32)
    perm = jnp.argsort(dst)
    dsts = dst[perm]
    srcs = src[perm]
    attr_s = edge_attr[perm].astype(jnp.int32)

    srcs_p = _pad_to(srcs, EPAD, 0)
    dsts_p = _pad_to(dsts, EPAD, 1 << 29)
    sd = jnp.stack([srcs_p.reshape(EPAD // CH, CH),
                    dsts_p.reshape(EPAD // CH, CH)], axis=1)

    qb = jnp.searchsorted(
        dsts, jnp.arange(0, 50001, NQ, dtype=jnp.int32)).astype(jnp.int32)
    w = jnp.arange(NS, dtype=jnp.int32)
    lo_q = qb[:NPART][:, None]
    hi_q = qb[1:][:, None]
    starts = (lo_q + (w[None, :] * (hi_q - lo_q)) // NS) & ~127
    ends = jnp.concatenate([starts[:, 1:], hi_q], axis=1)
    meta = jnp.concatenate(
        [jnp.stack([starts, ends], axis=1).astype(jnp.float32).reshape(
            NPART * 32),
         jnp.zeros((32,), jnp.float32)])
    meta = _pad_to(meta, 672, 0.0)

    xT = [_blocks3d(x[:, k].astype(jnp.int32), NBLK, 0) for k in range(4)]
    h = _enc_atom(*xT, atom_emb[0], atom_emb[1], atom_emb[2], atom_emb[3])
    aT = [_blocks3d(attr_s[:, k], EBLK, 0) for k in range(3)]
    ea_s = _enc_bond(*aT, bond_emb[0], bond_emb[1], bond_emb[2])

    ones64 = jnp.ones((EMB,), f32)
    zeros64 = jnp.zeros((EMB,), f32)
    A, B, lo = ones64, zeros64, 1.0

    for i in range(N_LAYERS):
        par = jnp.zeros((16, 16), f32)
        par = par.at[0:4, :].set(A.reshape(4, 16))
        par = par.at[4:8, :].set(B.reshape(4, 16))
        par = par.at[8, :].set(t[i])
        par = par.at[9, :].set(lo)
        acc4 = _sc_edge(h, ea_s, sd, meta, par)
        acc = acc4[:, :NQ, :].reshape(N_NODES, 128)
        rm = jnp.full((1, EMB), 0.0 if i == 0 else 1.0, jnp.float32)
        h, hst = _layer(acc, h, W1[i], b1[i].reshape(1, -1),
                        A.reshape(1, -1), B.reshape(1, -1),
                        jnp.full((1, EMB), lo, f32),
                        bn_g[i].reshape(1, -1), bn_b[i].reshape(1, -1),
                        W2[i], b2[i].reshape(1, -1), rm)
        muh = hst[0] / N_NODES
        varh = hst[1] / N_NODES - muh * muh
        j = i + 1 if i + 1 < N_LAYERS else 0
        A = ln_g[j] * lax.rsqrt(varh + EPS_BN)
        B = ln_b[j] - muh * A
        lo = 0.0

    b3d = _blocks3d(batch.astype(jnp.int32), NBLK, 1 << 29)
    Wp_pad = jnp.zeros((EMB, 128), f32).at[:, :Wp.shape[1]].set(Wp)
    bp_pad = jnp.zeros((1, 128), f32).at[0, :bp.shape[0]].set(bp)
    outp = _pool(h, b3d, A.reshape(1, -1), B.reshape(1, -1), Wp_pad, bp_pad)
    return outp[:, :Wp.shape[1]]
